# baseline (speedup 1.0000x reference)
import functools

import jax
import jax.numpy as jnp
import numpy as np
from jax import lax
from jax.experimental import pallas as pl
from jax.experimental.pallas import tpu as pltpu
from jax.experimental.pallas import tpu_sc as plsc

N = 131072
N_LEVELS = 16
F = 2
T = 1 << 20
BASE_RES = 16
SCALE = 1.4142135624
LC = N_LEVELS * 8

_P1 = np.uint32(2654435761).astype(np.int32).item()
_P2 = np.uint32(805459861).astype(np.int32).item()

_RES = [int(np.floor(BASE_RES * (SCALE ** l))) for l in range(N_LEVELS)]
_DENSE = [(r + 1) ** 3 <= T for r in _RES]

NC = 2
NS = 16
NW = NC * NS
P = 128
NCH = N // P
CH_W = NCH // NW


def _idxw_body(xt_ref, idx_ref, wgt_ref):
    x0 = xt_ref[0:1, :]
    x1 = xt_ref[1:2, :]
    x2 = xt_ref[2:3, :]
    for l in range(N_LEVELS):
        res = _RES[l]
        resf = float(res)
        p0 = []
        frac = []
        for xd in (x0, x1, x2):
            pos = xd * resf
            p0f = jnp.floor(pos)
            frac.append(pos - p0f)
            p0.append(p0f.astype(jnp.int32))
        if _DENSE[l]:
            s1 = res + 1
            s2 = s1 * s1
        for c in range(8):
            offs = ((c >> 2) & 1, (c >> 1) & 1, c & 1)
            cd = [jnp.clip(p0[d] + offs[d], 0, res) for d in range(3)]
            if _DENSE[l]:
                flat = cd[0] + cd[1] * s1 + cd[2] * s2
            else:
                flat = (cd[0] ^ (cd[1] * _P1) ^ (cd[2] * _P2)) & (T - 1)
            w = frac[0] if offs[0] == 1 else (1.0 - frac[0])
            for d in (1, 2):
                w = w * (frac[d] if offs[d] == 1 else (1.0 - frac[d]))
            row = l * 8 + c
            idx_ref[row:row + 1, :] = flat + l * T
            wgt_ref[row:row + 1, :] = w


def _compute_idx_wgt(xT, block):
    return pl.pallas_call(
        _idxw_body,
        grid=(N // block,),
        in_specs=[pl.BlockSpec((3, block), lambda i: (0, i))],
        out_specs=[
            pl.BlockSpec((LC, block), lambda i: (0, i)),
            pl.BlockSpec((LC, block), lambda i: (0, i)),
        ],
        out_shape=[
            jax.ShapeDtypeStruct((LC, N), jnp.int32),
            jax.ShapeDtypeStruct((LC, N), jnp.float32),
        ],
    )(xT)


def _dg(v, idx16):
    return lax.gather(
        v, idx16[:, None],
        lax.GatherDimensionNumbers(offset_dims=(), collapsed_slice_dims=(0,),
                                   start_index_map=(0,)),
        (1,), mode=lax.GatherScatterMode.PROMISE_IN_BOUNDS)


def _sc_body(tf_hbm, idx_hbm, wgt_hbm, enc_hbm, idx_v, idx2_v, wgt_v, rows_v,
             enc_v, sem):
    cid = lax.axis_index("c")
    sid = lax.axis_index("s")
    wid = sid * NC + cid

    def chunk_body(t, carry):
        cb = wid * CH_W + t
        pltpu.sync_copy(idx_hbm.at[:, cb, :], idx_v)
        pltpu.sync_copy(wgt_hbm.at[:, cb, :], wgt_v)

        def expand(g, carry2):
            o16 = g * 16
            for lc in range(LC):
                iv = idx_v[lc, pl.ds(o16, 16)]
                e0 = iv * 2
                idx2_v[2 * lc, pl.ds(o16, 16)] = e0
                idx2_v[2 * lc + 1, pl.ds(o16, 16)] = e0 + 1
            return carry2

        lax.fori_loop(0, P // 16, expand, 0)

        def fire(j, carry2):
            pltpu.async_copy(tf_hbm.at[idx2_v.at[j]], rows_v.at[j], sem)
            return carry2

        lax.fori_loop(0, 2 * LC, fire, 0, unroll=8)

        def drain(j, carry2):
            pltpu.make_async_copy(tf_hbm.at[idx2_v.at[0]], rows_v.at[0],
                                  sem).wait()
            return carry2

        lax.fori_loop(0, 2 * LC, drain, 0, unroll=8)

        def group(g, carry2):
            o16 = g * 16
            for l in range(N_LEVELS):
                acc0 = acc1 = None
                for c in range(8):
                    lc = l * 8 + c
                    v0 = rows_v[2 * lc, pl.ds(o16, 16)]
                    v1 = rows_v[2 * lc + 1, pl.ds(o16, 16)]
                    wv = wgt_v[lc, pl.ds(o16, 16)]
                    if acc0 is None:
                        acc0 = v0 * wv
                        acc1 = v1 * wv
                    else:
                        acc0 = acc0 + v0 * wv
                        acc1 = acc1 + v1 * wv
                enc_v[2 * l, pl.ds(o16, 16)] = acc0
                enc_v[2 * l + 1, pl.ds(o16, 16)] = acc1
            return carry2

        lax.fori_loop(0, P // 16, group, 0)
        pltpu.sync_copy(enc_v, enc_hbm.at[:, pl.ds(cb * P, P)])
        return carry

    lax.fori_loop(0, CH_W, chunk_body, 0)


def _sc_gather_interp(tf, idx3, wgt3):
    mesh = plsc.VectorSubcoreMesh(core_axis_name="c", subcore_axis_name="s")
    f = pl.kernel(
        _sc_body,
        out_type=jax.ShapeDtypeStruct((2 * N_LEVELS, N), jnp.float32),
        mesh=mesh,
        scratch_types=[
            pltpu.VMEM((LC, P), jnp.int32),
            pltpu.VMEM((F * LC, P), jnp.int32),
            pltpu.VMEM((LC, P), jnp.float32),
            pltpu.VMEM((F * LC, P), jnp.float32),
            pltpu.VMEM((2 * N_LEVELS, P), jnp.float32),
            pltpu.SemaphoreType.DMA,
        ],
    )
    return f(tf, idx3, wgt3)


def _mlp_body(enc_ref, w0t_ref, w1t_ref, w2t_ref, out_ref):
    h = jnp.maximum(
        lax.dot_general(w0t_ref[...], enc_ref[...], (((1,), (0,)), ((), ())),
                        preferred_element_type=jnp.float32), 0.0)
    h = jnp.maximum(
        lax.dot_general(w1t_ref[...], h, (((1,), (0,)), ((), ())),
                        preferred_element_type=jnp.float32), 0.0)
    o = lax.dot_general(w2t_ref[...], h, (((1,), (0,)), ((), ())),
                        preferred_element_type=jnp.float32)
    out_ref[...] = jnp.log1p(jnp.exp(-jnp.abs(o))) + jnp.maximum(o, 0.0)


def _mlp(enc, W0, W1, W2, block):
    W0T = W0.T
    W1T = W1.T
    W2T = jnp.zeros((8, 128), jnp.float32).at[:F].set(W2.T)
    return pl.pallas_call(
        _mlp_body,
        grid=(N // block,),
        in_specs=[
            pl.BlockSpec((2 * N_LEVELS, block), lambda i: (0, i)),
            pl.BlockSpec((128, 32), lambda i: (0, 0)),
            pl.BlockSpec((128, 128), lambda i: (0, 0)),
            pl.BlockSpec((8, 128), lambda i: (0, 0)),
        ],
        out_specs=pl.BlockSpec((8, block), lambda i: (0, i)),
        out_shape=jax.ShapeDtypeStruct((8, N), jnp.float32),
    )(enc, W0T, W1T, W2T)


def kernel(x, tables, W0, W1, W2):
    xT = x.T
    idx, wgt = _compute_idx_wgt(xT, block=2048)
    tf = tables.reshape(N_LEVELS * T * F)
    idx3 = idx.reshape(LC, NCH, P)
    wgt3 = wgt.reshape(LC, NCH, P)
    enc = _sc_gather_interp(tf, idx3, wgt3)
    out = _mlp(enc, W0, W1, W2, block=2048)
    return (out[0], out[1])


# speedup vs baseline: 13.5357x; 13.5357x over previous
import functools

import jax
import jax.numpy as jnp
import numpy as np
from jax import lax
from jax.experimental import pallas as pl
from jax.experimental.pallas import tpu as pltpu
from jax.experimental.pallas import tpu_sc as plsc

N = 131072
N_LEVELS = 16
F = 2
T = 1 << 20
BASE_RES = 16
SCALE = 1.4142135624
LC = N_LEVELS * 8

_P1 = np.uint32(2654435761).astype(np.int32).item()
_P2 = np.uint32(805459861).astype(np.int32).item()

_RES = [int(np.floor(BASE_RES * (SCALE ** l))) for l in range(N_LEVELS)]
_DENSE = [(r + 1) ** 3 <= T for r in _RES]

NC = 2
NS = 16
NW = NC * NS
P = 128
NCH = N // P
CH_W = NCH // NW


def _idxw_body(xt_ref, idx_ref, wgt_ref):
    x0 = xt_ref[0:1, :]
    x1 = xt_ref[1:2, :]
    x2 = xt_ref[2:3, :]
    for l in range(N_LEVELS):
        res = _RES[l]
        resf = float(res)
        p0 = []
        frac = []
        for xd in (x0, x1, x2):
            pos = xd * resf
            p0f = jnp.floor(pos)
            frac.append(pos - p0f)
            p0.append(p0f.astype(jnp.int32))
        if _DENSE[l]:
            s1 = res + 1
            s2 = s1 * s1
        for c in range(8):
            offs = ((c >> 2) & 1, (c >> 1) & 1, c & 1)
            cd = [jnp.clip(p0[d] + offs[d], 0, res) for d in range(3)]
            if _DENSE[l]:
                flat = cd[0] + cd[1] * s1 + cd[2] * s2
            else:
                flat = (cd[0] ^ (cd[1] * _P1) ^ (cd[2] * _P2)) & (T - 1)
            w = frac[0] if offs[0] == 1 else (1.0 - frac[0])
            for d in (1, 2):
                w = w * (frac[d] if offs[d] == 1 else (1.0 - frac[d]))
            row = l * 8 + c
            e0 = l * (2 * T) + (flat >> 7) * 256 + (flat & 127)
            idx_ref[row:row + 1, :] = e0
            wgt_ref[row:row + 1, :] = w


def _compute_idx_wgt(xT, block):
    return pl.pallas_call(
        _idxw_body,
        grid=(N // block,),
        in_specs=[pl.BlockSpec((3, block), lambda i: (0, i))],
        out_specs=[
            pl.BlockSpec((LC, block), lambda i: (0, i)),
            pl.BlockSpec((LC, block), lambda i: (0, i)),
        ],
        out_shape=[
            jax.ShapeDtypeStruct((LC, N), jnp.int32),
            jax.ShapeDtypeStruct((LC, N), jnp.float32),
        ],
    )(xT)


def _dg(v, idx16):
    return lax.gather(
        v, idx16[:, None],
        lax.GatherDimensionNumbers(offset_dims=(), collapsed_slice_dims=(0,),
                                   start_index_map=(0,)),
        (1,), mode=lax.GatherScatterMode.PROMISE_IN_BOUNDS)


def _sc_body(tf_hbm, idx_hbm, wgt_hbm, enc_hbm, idx_v, ch_iv, wgt_v, rows_v,
             enc_v, sem):
    cid = lax.axis_index("c")
    sid = lax.axis_index("s")
    wid = sid * NC + cid

    def chunk_body(t, carry):
        cb = wid * CH_W + t
        pltpu.sync_copy(idx_hbm.at[:, pl.ds(cb * P, P)], idx_v)
        pltpu.sync_copy(wgt_hbm.at[:, pl.ds(cb * P, P)], wgt_v)

        def expand(g, carry2):
            o16 = g * 16
            for lc in range(LC):
                iv = idx_v[lc, pl.ds(o16, 16)]
                ch_iv[lc, pl.ds(o16, 16)] = iv + 128
            return carry2

        lax.fori_loop(0, P // 16, expand, 0)

        def fire(j, carry2):
            pltpu.async_copy(tf_hbm.at[idx_v.at[j]], rows_v.at[2 * j], sem)
            pltpu.async_copy(tf_hbm.at[ch_iv.at[j]], rows_v.at[2 * j + 1],
                             sem)
            return carry2

        lax.fori_loop(0, LC, fire, 0, unroll=8)

        def drain(j, carry2):
            pltpu.make_async_copy(tf_hbm.at[idx_v.at[0]], rows_v.at[0],
                                  sem).wait()
            return carry2

        lax.fori_loop(0, 2 * LC, drain, 0, unroll=8)

        def group(g, carry2):
            o16 = g * 16
            for l in range(N_LEVELS):
                acc0 = acc1 = None
                for c in range(8):
                    lc = l * 8 + c
                    v0 = rows_v[2 * lc, pl.ds(o16, 16)]
                    v1 = rows_v[2 * lc + 1, pl.ds(o16, 16)]
                    wv = wgt_v[lc, pl.ds(o16, 16)]
                    if acc0 is None:
                        acc0 = v0 * wv
                        acc1 = v1 * wv
                    else:
                        acc0 = acc0 + v0 * wv
                        acc1 = acc1 + v1 * wv
                enc_v[2 * l, pl.ds(o16, 16)] = acc0
                enc_v[2 * l + 1, pl.ds(o16, 16)] = acc1
            return carry2

        lax.fori_loop(0, P // 16, group, 0)
        pltpu.sync_copy(enc_v, enc_hbm.at[:, pl.ds(cb * P, P)])
        return carry

    lax.fori_loop(0, CH_W, chunk_body, 0)


def _sc_gather_interp(tf, idx3, wgt3):
    mesh = plsc.VectorSubcoreMesh(core_axis_name="c", subcore_axis_name="s")
    f = pl.kernel(
        _sc_body,
        out_type=jax.ShapeDtypeStruct((2 * N_LEVELS, N), jnp.float32),
        mesh=mesh,
        scratch_types=[
            pltpu.VMEM((LC, P), jnp.int32),
            pltpu.VMEM((LC, P), jnp.int32),
            pltpu.VMEM((LC, P), jnp.float32),
            pltpu.VMEM((F * LC, P), jnp.float32),
            pltpu.VMEM((2 * N_LEVELS, P), jnp.float32),
            pltpu.SemaphoreType.DMA,
        ],
    )
    return f(tf, idx3, wgt3)


def _mlp_body(enc_ref, w0t_ref, w1t_ref, w2t_ref, out_ref):
    h = jnp.maximum(
        lax.dot_general(w0t_ref[...], enc_ref[...], (((1,), (0,)), ((), ())),
                        preferred_element_type=jnp.float32), 0.0)
    h = jnp.maximum(
        lax.dot_general(w1t_ref[...], h, (((1,), (0,)), ((), ())),
                        preferred_element_type=jnp.float32), 0.0)
    o = lax.dot_general(w2t_ref[...], h, (((1,), (0,)), ((), ())),
                        preferred_element_type=jnp.float32)
    out_ref[...] = jnp.log1p(jnp.exp(-jnp.abs(o))) + jnp.maximum(o, 0.0)


def _mlp(enc, W0, W1, W2, block):
    W0T = W0.T
    W1T = W1.T
    W2T = jnp.zeros((8, 128), jnp.float32).at[:F].set(W2.T)
    return pl.pallas_call(
        _mlp_body,
        grid=(N // block,),
        in_specs=[
            pl.BlockSpec((2 * N_LEVELS, block), lambda i: (0, i)),
            pl.BlockSpec((128, 32), lambda i: (0, 0)),
            pl.BlockSpec((128, 128), lambda i: (0, 0)),
            pl.BlockSpec((8, 128), lambda i: (0, 0)),
        ],
        out_specs=pl.BlockSpec((8, block), lambda i: (0, i)),
        out_shape=jax.ShapeDtypeStruct((8, N), jnp.float32),
    )(enc, W0T, W1T, W2T)


def kernel(x, tables, W0, W1, W2):
    xT = x.T
    idx, wgt = _compute_idx_wgt(xT, block=2048)
    tf = tables.reshape(N_LEVELS, T // 128, 128, F)
    tf = tf.transpose(0, 1, 3, 2).reshape(N_LEVELS * T * F)
    enc = _sc_gather_interp(tf, idx, wgt)
    out = _mlp(enc, W0, W1, W2, block=2048)
    return (out[0], out[1])


# speedup vs baseline: 15.5741x; 1.1506x over previous
import functools

import jax
import jax.numpy as jnp
import numpy as np
from jax import lax
from jax.experimental import pallas as pl
from jax.experimental.pallas import tpu as pltpu
from jax.experimental.pallas import tpu_sc as plsc

N = 131072
N_LEVELS = 16
F = 2
T = 1 << 20
BASE_RES = 16
SCALE = 1.4142135624
LC = N_LEVELS * 8

_P1 = np.uint32(2654435761).astype(np.int32).item()
_P2 = np.uint32(805459861).astype(np.int32).item()

_RES = [int(np.floor(BASE_RES * (SCALE ** l))) for l in range(N_LEVELS)]
_DENSE = [(r + 1) ** 3 <= T for r in _RES]

NC = 2
NS = 16
NW = NC * NS
P = 128
CH_W = N // (NW * P)
HL = LC // 2


def _idxw_body(xt_ref, idx_ref, wgt_ref):
    x0 = xt_ref[0:1, :]
    x1 = xt_ref[1:2, :]
    x2 = xt_ref[2:3, :]
    for l in range(N_LEVELS):
        res = _RES[l]
        resf = float(res)
        p0 = []
        frac = []
        for xd in (x0, x1, x2):
            pos = xd * resf
            p0f = jnp.floor(pos)
            frac.append(pos - p0f)
            p0.append(p0f.astype(jnp.int32))
        if _DENSE[l]:
            s1 = res + 1
            s2 = s1 * s1
        for c in range(8):
            offs = ((c >> 2) & 1, (c >> 1) & 1, c & 1)
            cd = [jnp.clip(p0[d] + offs[d], 0, res) for d in range(3)]
            if _DENSE[l]:
                flat = cd[0] + cd[1] * s1 + cd[2] * s2
            else:
                flat = (cd[0] ^ (cd[1] * _P1) ^ (cd[2] * _P2)) & (T - 1)
            w = frac[0] if offs[0] == 1 else (1.0 - frac[0])
            for d in (1, 2):
                w = w * (frac[d] if offs[d] == 1 else (1.0 - frac[d]))
            row = l * 8 + c
            e0 = l * (2 * T) + (flat >> 7) * 256 + (flat & 127)
            idx_ref[row:row + 1, :] = e0
            wgt_ref[row:row + 1, :] = w


def _compute_idx_wgt(xT, block):
    return pl.pallas_call(
        _idxw_body,
        grid=(N // block,),
        in_specs=[pl.BlockSpec((3, block), lambda i: (0, i))],
        out_specs=[
            pl.BlockSpec((LC, block), lambda i: (0, i)),
            pl.BlockSpec((LC, block), lambda i: (0, i)),
        ],
        out_shape=[
            jax.ShapeDtypeStruct((LC, N), jnp.int32),
            jax.ShapeDtypeStruct((LC, N), jnp.float32),
        ],
    )(xT)


def _dg(v, idx16):
    return lax.gather(
        v, idx16[:, None],
        lax.GatherDimensionNumbers(offset_dims=(), collapsed_slice_dims=(0,),
                                   start_index_map=(0,)),
        (1,), mode=lax.GatherScatterMode.PROMISE_IN_BOUNDS)


def _sc_body(tf_hbm, idx_hbm, wgt_hbm, enc_hbm, i2a, i2b, wgt_v, ra, rb,
             enc_v, sema, semb):
    cid = lax.axis_index("c")
    sid = lax.axis_index("s")
    wid = sid * NC + cid
    base = wid * CH_W

    def start(u, i2, sem, rv):
        cb = base + (u // 2)
        h = u % 2
        pltpu.sync_copy(
            idx_hbm.at[pl.ds(h * HL, HL), pl.ds(cb * P, P)],
            i2.at[:, pl.ds(0, P)])

        def expand(g, carry2):
            o16 = g * 16
            for lc in range(HL):
                i2[lc, pl.ds(P + o16, 16)] = i2[lc, pl.ds(o16, 16)] + 128
            return carry2

        lax.fori_loop(0, P // 16, expand, 0)

        def fire(j, carry2):
            pltpu.async_copy(tf_hbm.at[i2.at[j, pl.ds(0, P)]],
                             rv.at[j, pl.ds(0, P)], sem)
            pltpu.async_copy(tf_hbm.at[i2.at[j, pl.ds(P, P)]],
                             rv.at[j, pl.ds(P, P)], sem)
            return carry2

        lax.fori_loop(0, HL, fire, 0, unroll=8)

    def finish(u, i2, sem, rv):
        cb = base + (u // 2)
        h = u % 2

        def drain(j, carry2):
            pltpu.make_async_copy(tf_hbm.at[i2.at[0, pl.ds(0, P)]],
                                  rv.at[0, pl.ds(0, P)], sem).wait()
            return carry2

        lax.fori_loop(0, 2 * HL, drain, 0, unroll=8)
        pltpu.sync_copy(
            wgt_hbm.at[pl.ds(h * HL, HL), pl.ds(cb * P, P)], wgt_v)

        def group(g, carry2):
            o16 = g * 16
            for ll in range(N_LEVELS // 2):
                acc0 = acc1 = None
                for c in range(8):
                    lc = ll * 8 + c
                    v0 = rv[lc, pl.ds(o16, 16)]
                    v1 = rv[lc, pl.ds(P + o16, 16)]
                    wv = wgt_v[lc, pl.ds(o16, 16)]
                    if acc0 is None:
                        acc0 = v0 * wv
                        acc1 = v1 * wv
                    else:
                        acc0 = acc0 + v0 * wv
                        acc1 = acc1 + v1 * wv
                enc_v[2 * ll, pl.ds(o16, 16)] = acc0
                enc_v[2 * ll + 1, pl.ds(o16, 16)] = acc1
            return carry2

        lax.fori_loop(0, P // 16, group, 0)
        pltpu.sync_copy(
            enc_v,
            enc_hbm.at[pl.ds(h * N_LEVELS, N_LEVELS), pl.ds(cb * P, P)])

    NU = 2 * CH_W
    start(0, i2a, sema, ra)

    def pair_body(i, carry):
        u0 = 2 * i
        start(u0 + 1, i2b, semb, rb)
        finish(u0, i2a, sema, ra)

        @pl.when(u0 + 2 < NU)
        def _():
            start(u0 + 2, i2a, sema, ra)

        finish(u0 + 1, i2b, semb, rb)
        return carry

    lax.fori_loop(0, NU // 2, pair_body, 0)


def _sc_gather_interp(tf, idx3, wgt3):
    mesh = plsc.VectorSubcoreMesh(core_axis_name="c", subcore_axis_name="s")
    f = pl.kernel(
        _sc_body,
        out_type=jax.ShapeDtypeStruct((2 * N_LEVELS, N), jnp.float32),
        mesh=mesh,
        scratch_types=[
            pltpu.VMEM((HL, F * P), jnp.int32),
            pltpu.VMEM((HL, F * P), jnp.int32),
            pltpu.VMEM((HL, P), jnp.float32),
            pltpu.VMEM((HL, F * P), jnp.float32),
            pltpu.VMEM((HL, F * P), jnp.float32),
            pltpu.VMEM((N_LEVELS, P), jnp.float32),
            pltpu.SemaphoreType.DMA,
            pltpu.SemaphoreType.DMA,
        ],
    )
    return f(tf, idx3, wgt3)


def _mlp_body(enc_ref, w0t_ref, w1t_ref, w2t_ref, out_ref):
    h = jnp.maximum(
        lax.dot_general(w0t_ref[...], enc_ref[...], (((1,), (0,)), ((), ())),
                        preferred_element_type=jnp.float32), 0.0)
    h = jnp.maximum(
        lax.dot_general(w1t_ref[...], h, (((1,), (0,)), ((), ())),
                        preferred_element_type=jnp.float32), 0.0)
    o = lax.dot_general(w2t_ref[...], h, (((1,), (0,)), ((), ())),
                        preferred_element_type=jnp.float32)
    out_ref[...] = jnp.log1p(jnp.exp(-jnp.abs(o))) + jnp.maximum(o, 0.0)


def _mlp(enc, W0, W1, W2, block):
    W0T = W0.T
    W1T = W1.T
    W2T = jnp.zeros((8, 128), jnp.float32).at[:F].set(W2.T)
    return pl.pallas_call(
        _mlp_body,
        grid=(N // block,),
        in_specs=[
            pl.BlockSpec((2 * N_LEVELS, block), lambda i: (0, i)),
            pl.BlockSpec((128, 32), lambda i: (0, 0)),
            pl.BlockSpec((128, 128), lambda i: (0, 0)),
            pl.BlockSpec((8, 128), lambda i: (0, 0)),
        ],
        out_specs=pl.BlockSpec((8, block), lambda i: (0, i)),
        out_shape=jax.ShapeDtypeStruct((8, N), jnp.float32),
    )(enc, W0T, W1T, W2T)


def kernel(x, tables, W0, W1, W2):
    xT = x.T
    idx, wgt = _compute_idx_wgt(xT, block=2048)
    tf = tables.reshape(N_LEVELS, T // 128, 128, F)
    tf = tf.transpose(0, 1, 3, 2).reshape(N_LEVELS * T * F)
    enc = _sc_gather_interp(tf, idx, wgt)
    out = _mlp(enc, W0, W1, W2, block=2048)
    return (out[0], out[1])


# speedup vs baseline: 15.5940x; 1.0013x over previous
import functools

import jax
import jax.numpy as jnp
import numpy as np
from jax import lax
from jax.experimental import pallas as pl
from jax.experimental.pallas import tpu as pltpu
from jax.experimental.pallas import tpu_sc as plsc

N = 131072
N_LEVELS = 16
F = 2
T = 1 << 20
BASE_RES = 16
SCALE = 1.4142135624
LC = N_LEVELS * 8

_P1 = np.uint32(2654435761).astype(np.int32).item()
_P2 = np.uint32(805459861).astype(np.int32).item()

_RES = [int(np.floor(BASE_RES * (SCALE ** l))) for l in range(N_LEVELS)]
_DENSE = [(r + 1) ** 3 <= T for r in _RES]

NC = 2
NS = 16
NW = NC * NS
P = 128
CH_W = N // (NW * P)
HL = LC // 2


def _idxw_body(xt_ref, idx_ref, wgt_ref):
    x0 = xt_ref[0:1, :]
    x1 = xt_ref[1:2, :]
    x2 = xt_ref[2:3, :]
    for l in range(N_LEVELS):
        res = _RES[l]
        resf = float(res)
        p0 = []
        frac = []
        for xd in (x0, x1, x2):
            pos = xd * resf
            p0f = jnp.floor(pos)
            frac.append(pos - p0f)
            p0.append(p0f.astype(jnp.int32))
        if _DENSE[l]:
            s1 = res + 1
            s2 = s1 * s1
        for c in range(8):
            offs = ((c >> 2) & 1, (c >> 1) & 1, c & 1)
            cd = [jnp.clip(p0[d] + offs[d], 0, res) for d in range(3)]
            if _DENSE[l]:
                flat = cd[0] + cd[1] * s1 + cd[2] * s2
            else:
                flat = (cd[0] ^ (cd[1] * _P1) ^ (cd[2] * _P2)) & (T - 1)
            w = frac[0] if offs[0] == 1 else (1.0 - frac[0])
            for d in (1, 2):
                w = w * (frac[d] if offs[d] == 1 else (1.0 - frac[d]))
            row = l * 8 + c
            e0 = l * (2 * T) + (flat >> 7) * 256 + (flat & 127)
            idx_ref[row:row + 1, :] = e0
            wgt_ref[row:row + 1, :] = w


def _compute_idx_wgt(xT, block):
    return pl.pallas_call(
        _idxw_body,
        grid=(N // block,),
        in_specs=[pl.BlockSpec((3, block), lambda i: (0, i))],
        out_specs=[
            pl.BlockSpec((LC, block), lambda i: (0, i)),
            pl.BlockSpec((LC, block), lambda i: (0, i)),
        ],
        out_shape=[
            jax.ShapeDtypeStruct((LC, N), jnp.int32),
            jax.ShapeDtypeStruct((LC, N), jnp.float32),
        ],
    )(xT)


def _dg(v, idx16):
    return lax.gather(
        v, idx16[:, None],
        lax.GatherDimensionNumbers(offset_dims=(), collapsed_slice_dims=(0,),
                                   start_index_map=(0,)),
        (1,), mode=lax.GatherScatterMode.PROMISE_IN_BOUNDS)


def _sc_body(tf_hbm, idx_hbm, wgt_hbm, enc_hbm, i2a, i2b, wgt_v, ra, rb,
             enc_v, sema, semb):
    cid = lax.axis_index("c")
    sid = lax.axis_index("s")
    wid = sid * NC + cid
    base = wid * CH_W

    def start(u, i2, sem, rv):
        cb = base + (u // 2)
        h = u % 2
        pltpu.sync_copy(
            idx_hbm.at[pl.ds(h * HL, HL), pl.ds(cb * P, P)],
            i2.at[:, pl.ds(0, P)])

        def expand(g, carry2):
            o16 = g * 16
            for lc in range(HL):
                i2[lc, pl.ds(P + o16, 16)] = i2[lc, pl.ds(o16, 16)] + 128
            return carry2

        lax.fori_loop(0, P // 16, expand, 0)

        def fire(j, carry2):
            pltpu.async_copy(tf_hbm.at[i2.at[j, pl.ds(0, P)]],
                             rv.at[j, pl.ds(0, P)], sem)
            pltpu.async_copy(tf_hbm.at[i2.at[j, pl.ds(P, P)]],
                             rv.at[j, pl.ds(P, P)], sem)
            return carry2

        lax.fori_loop(0, HL, fire, 0, unroll=8)

    def finish(u, i2, sem, rv):
        cb = base + (u // 2)
        h = u % 2

        pltpu.make_async_copy(
            wgt_hbm.at[pl.ds(0, HL), pl.ds(0, F * P)], rv, sem).wait()
        pltpu.sync_copy(
            wgt_hbm.at[pl.ds(h * HL, HL), pl.ds(cb * P, P)], wgt_v)

        def group(g, carry2):
            o16 = g * 16
            for ll in range(N_LEVELS // 2):
                acc0 = acc1 = None
                for c in range(8):
                    lc = ll * 8 + c
                    v0 = rv[lc, pl.ds(o16, 16)]
                    v1 = rv[lc, pl.ds(P + o16, 16)]
                    wv = wgt_v[lc, pl.ds(o16, 16)]
                    if acc0 is None:
                        acc0 = v0 * wv
                        acc1 = v1 * wv
                    else:
                        acc0 = acc0 + v0 * wv
                        acc1 = acc1 + v1 * wv
                enc_v[2 * ll, pl.ds(o16, 16)] = acc0
                enc_v[2 * ll + 1, pl.ds(o16, 16)] = acc1
            return carry2

        lax.fori_loop(0, P // 16, group, 0)
        pltpu.sync_copy(
            enc_v,
            enc_hbm.at[pl.ds(h * N_LEVELS, N_LEVELS), pl.ds(cb * P, P)])

    NU = 2 * CH_W
    start(0, i2a, sema, ra)

    def pair_body(i, carry):
        u0 = 2 * i
        start(u0 + 1, i2b, semb, rb)
        finish(u0, i2a, sema, ra)

        @pl.when(u0 + 2 < NU)
        def _():
            start(u0 + 2, i2a, sema, ra)

        finish(u0 + 1, i2b, semb, rb)
        return carry

    lax.fori_loop(0, NU // 2, pair_body, 0)


def _sc_gather_interp(tf, idx3, wgt3):
    mesh = plsc.VectorSubcoreMesh(core_axis_name="c", subcore_axis_name="s")
    f = pl.kernel(
        _sc_body,
        out_type=jax.ShapeDtypeStruct((2 * N_LEVELS, N), jnp.float32),
        mesh=mesh,
        scratch_types=[
            pltpu.VMEM((HL, F * P), jnp.int32),
            pltpu.VMEM((HL, F * P), jnp.int32),
            pltpu.VMEM((HL, P), jnp.float32),
            pltpu.VMEM((HL, F * P), jnp.float32),
            pltpu.VMEM((HL, F * P), jnp.float32),
            pltpu.VMEM((N_LEVELS, P), jnp.float32),
            pltpu.SemaphoreType.DMA,
            pltpu.SemaphoreType.DMA,
        ],
    )
    return f(tf, idx3, wgt3)


def _mlp_body(enc_ref, w0t_ref, w1t_ref, w2t_ref, out_ref):
    h = jnp.maximum(
        lax.dot_general(w0t_ref[...], enc_ref[...], (((1,), (0,)), ((), ())),
                        preferred_element_type=jnp.float32), 0.0)
    h = jnp.maximum(
        lax.dot_general(w1t_ref[...], h, (((1,), (0,)), ((), ())),
                        preferred_element_type=jnp.float32), 0.0)
    o = lax.dot_general(w2t_ref[...], h, (((1,), (0,)), ((), ())),
                        preferred_element_type=jnp.float32)
    out_ref[...] = jnp.log1p(jnp.exp(-jnp.abs(o))) + jnp.maximum(o, 0.0)


def _mlp(enc, W0, W1, W2, block):
    W0T = W0.T
    W1T = W1.T
    W2T = jnp.zeros((8, 128), jnp.float32).at[:F].set(W2.T)
    return pl.pallas_call(
        _mlp_body,
        grid=(N // block,),
        in_specs=[
            pl.BlockSpec((2 * N_LEVELS, block), lambda i: (0, i)),
            pl.BlockSpec((128, 32), lambda i: (0, 0)),
            pl.BlockSpec((128, 128), lambda i: (0, 0)),
            pl.BlockSpec((8, 128), lambda i: (0, 0)),
        ],
        out_specs=pl.BlockSpec((8, block), lambda i: (0, i)),
        out_shape=jax.ShapeDtypeStruct((8, N), jnp.float32),
    )(enc, W0T, W1T, W2T)


def kernel(x, tables, W0, W1, W2):
    xT = x.T
    idx, wgt = _compute_idx_wgt(xT, block=2048)
    tf = tables.reshape(N_LEVELS, T // 128, 128, F)
    tf = tf.transpose(0, 1, 3, 2).reshape(N_LEVELS * T * F)
    enc = _sc_gather_interp(tf, idx, wgt)
    out = _mlp(enc, W0, W1, W2, block=2048)
    return (out[0], out[1])
